# Initial kernel scaffold; baseline (speedup 1.0000x reference)
import jax
import jax.numpy as jnp
from jax.experimental import pallas as pl


def kernel(voxel_features, indices):
    raise NotImplementedError("write your pallas kernel here")


# speedup vs baseline: 6.5915x; 6.5915x over previous
import functools

import jax
import jax.numpy as jnp
from jax import lax
from jax.experimental import pallas as pl
from jax.experimental.pallas import tpu as pltpu
from jax.experimental.pallas import tpu_sc as plsc

_G = 128
_CH = 32
_ROWS = _G * _G * _G
_PAD_ROWS = _ROWS + 8
_N = 150000
_CHUNK = 2048
_NCHUNK = 74
_NPAD = _CHUNK * _NCHUNK
_ZB = 512
_NSUB = 16
_ZPER = _ROWS // _NSUB
_IDXW = 128


def _build_sc_call():
  mesh = plsc.VectorSubcoreMesh(core_axis_name="c", subcore_axis_name="s")

  @functools.partial(
      pl.kernel,
      out_type=jax.ShapeDtypeStruct((_PAD_ROWS, _CH), jnp.float32),
      mesh=mesh,
      compiler_params=pltpu.CompilerParams(use_tc_tiling_on_sc=False),
      scratch_types=[
          pltpu.VMEM((_ZB, _CH), jnp.float32),
          pltpu.VMEM((_CHUNK,), jnp.int32),
          pltpu.VMEM((_CHUNK,), jnp.int32),
          pltpu.VMEM((_CHUNK,), jnp.int32),
          pltpu.VMEM((_IDXW,), jnp.int32),
          pltpu.VMEM((_CHUNK, _CH), jnp.float32),
          pltpu.SemaphoreType.DMA,
      ],
  )
  def grid_scatter(iz_hbm, iy_hbm, ix_hbm, f_hbm, out_hbm,
                   zbuf, izv, iyv, ixv, linv, fv, sem):
    cid = lax.axis_index("c")
    sid = lax.axis_index("s")

    @pl.when(cid == 0)
    def _core0():
      zeros16 = jnp.zeros((16,), jnp.float32)

      def _zrow(r, carry):
        zbuf[r, pl.ds(0, 16)] = zeros16
        zbuf[r, pl.ds(16, 16)] = zeros16
        return carry

      lax.fori_loop(0, _ZB, _zrow, 0)

      base = sid * _ZPER

      def _zdma(t, carry):
        pltpu.sync_copy(zbuf, out_hbm.at[pl.ds(base + t * _ZB, _ZB)])
        return carry

      lax.fori_loop(0, _ZPER // _ZB, _zdma, 0)

      plsc.subcore_barrier()

      @pl.when(sid == 0)
      def _scatter():
        def _chunk(c, carry):
          off = c * _CHUNK
          pltpu.sync_copy(iz_hbm.at[pl.ds(off, _CHUNK)], izv)
          pltpu.sync_copy(iy_hbm.at[pl.ds(off, _CHUNK)], iyv)
          pltpu.sync_copy(ix_hbm.at[pl.ds(off, _CHUNK)], ixv)
          pltpu.sync_copy(f_hbm.at[pl.ds(off, _CHUNK)], fv)

          for q in range(_CHUNK // _IDXW):
            def _lin(j, carry2, q=q):
              z16 = izv[pl.ds(q * _IDXW + j * 16, 16)]
              y16 = iyv[pl.ds(q * _IDXW + j * 16, 16)]
              x16 = ixv[pl.ds(q * _IDXW + j * 16, 16)]
              lin = z16 * (_G * _G) + y16 * _G + x16
              linv[pl.ds(j * 16, 16)] = lin
              return carry2

            lax.fori_loop(0, _IDXW // 16, _lin, 0)
            pltpu.async_copy(
                fv.at[pl.ds(q * _IDXW, _IDXW)], out_hbm.at[linv], sem).wait()
          return carry

        lax.fori_loop(0, _NCHUNK, _chunk, 0)

  return grid_scatter


_SC_CALL = _build_sc_call()


def kernel(voxel_features, indices):
  idx = indices.astype(jnp.int32)
  npad = _NPAD - _N
  iz = jnp.concatenate([idx[:, 0], jnp.full((npad,), _G, jnp.int32)])
  iy = jnp.concatenate([idx[:, 1], jnp.zeros((npad,), jnp.int32)])
  ix = jnp.concatenate([idx[:, 2], jnp.zeros((npad,), jnp.int32)])
  feats = jnp.concatenate(
      [voxel_features.astype(jnp.float32),
       jnp.zeros((npad, _CH), jnp.float32)])
  out = _SC_CALL(iz, iy, ix, feats)
  return out[:_ROWS].reshape(_G, _G, _G, _CH)


# speedup vs baseline: 7.7176x; 1.1708x over previous
import functools

import jax
import jax.numpy as jnp
from jax import lax
from jax.experimental import pallas as pl
from jax.experimental.pallas import tpu as pltpu
from jax.experimental.pallas import tpu_sc as plsc

_G = 128
_CH = 32
_ROWS = _G * _G * _G
_PAD_ROWS = _ROWS + 8
_N = 150000
_CHUNK = 1024
_NCHUNK = 148
_NPAD = _CHUNK * _NCHUNK
_ZB = 512
_NSUB = 16
_ZPER = _ROWS // _NSUB
_IDXW = 128
_NB = _CHUNK // _IDXW


def _build_sc_call():
  mesh = plsc.VectorSubcoreMesh(core_axis_name="c", subcore_axis_name="s")

  @functools.partial(
      pl.kernel,
      out_type=jax.ShapeDtypeStruct((_PAD_ROWS, _CH), jnp.float32),
      mesh=mesh,
      compiler_params=pltpu.CompilerParams(use_tc_tiling_on_sc=False),
      scratch_types=[
          pltpu.VMEM((_ZB, _CH), jnp.float32),
          [pltpu.VMEM((_CHUNK,), jnp.int32) for _ in range(2)],
          [pltpu.VMEM((_CHUNK,), jnp.int32) for _ in range(2)],
          [pltpu.VMEM((_CHUNK,), jnp.int32) for _ in range(2)],
          [pltpu.VMEM((_CHUNK, _CH), jnp.float32) for _ in range(2)],
          [pltpu.VMEM((_IDXW,), jnp.int32) for _ in range(_NB)],
          pltpu.SemaphoreType.DMA,
          [pltpu.SemaphoreType.DMA for _ in range(2)],
          pltpu.SemaphoreType.DMA,
      ],
  )
  def grid_scatter(iz_hbm, iy_hbm, ix_hbm, f_hbm, out_hbm,
                   zbuf, izv, iyv, ixv, fv, linv, zsem, isem, ssem):
    cid = lax.axis_index("c")
    sid = lax.axis_index("s")

    @pl.when(cid == 0)
    def _core0():
      zeros16 = jnp.zeros((16,), jnp.float32)

      def _zrow(r, carry):
        zbuf[r, pl.ds(0, 16)] = zeros16
        zbuf[r, pl.ds(16, 16)] = zeros16
        return carry

      lax.fori_loop(0, _ZB, _zrow, 0)

      base = sid * _ZPER

      def _zdma(t, carry):
        hs = [
            pltpu.async_copy(
                zbuf, out_hbm.at[pl.ds(base + (t * 8 + k) * _ZB, _ZB)], zsem)
            for k in range(8)
        ]
        for h in hs:
          h.wait()
        return carry

      lax.fori_loop(0, _ZPER // _ZB // 8, _zdma, 0)

      plsc.subcore_barrier()

      @pl.when(sid == 0)
      def _scatter():
        def _startin(b, c):
          off = c * _CHUNK
          pltpu.async_copy(iz_hbm.at[pl.ds(off, _CHUNK)], izv[b], isem[b])
          pltpu.async_copy(iy_hbm.at[pl.ds(off, _CHUNK)], iyv[b], isem[b])
          pltpu.async_copy(ix_hbm.at[pl.ds(off, _CHUNK)], ixv[b], isem[b])
          pltpu.async_copy(f_hbm.at[pl.ds(off, _CHUNK)], fv[b], isem[b])

        def _waitin(b):
          pltpu.make_async_copy(
              iz_hbm.at[pl.ds(0, _CHUNK)], izv[b], isem[b]).wait()
          pltpu.make_async_copy(
              iy_hbm.at[pl.ds(0, _CHUNK)], iyv[b], isem[b]).wait()
          pltpu.make_async_copy(
              ix_hbm.at[pl.ds(0, _CHUNK)], ixv[b], isem[b]).wait()
          pltpu.make_async_copy(
              f_hbm.at[pl.ds(0, _CHUNK)], fv[b], isem[b]).wait()

        def _scatter_chunk(b):
          for q in range(_NB):
            for j in range(_IDXW // 16):
              s = q * _IDXW + j * 16
              z16 = izv[b][pl.ds(s, 16)]
              y16 = iyv[b][pl.ds(s, 16)]
              x16 = ixv[b][pl.ds(s, 16)]
              linv[q][pl.ds(j * 16, 16)] = z16 * (_G * _G) + y16 * _G + x16
          for q in range(_NB):
            pltpu.async_copy(
                fv[b].at[pl.ds(q * _IDXW, _IDXW)],
                out_hbm.at[linv[q]], ssem).wait()

        _startin(0, 0)

        def _pair(cc, carry):
          c0 = cc * 2
          _waitin(0)
          _startin(1, c0 + 1)
          _scatter_chunk(0)
          _waitin(1)
          _startin(0, jnp.minimum(c0 + 2, _NCHUNK - 1))
          _scatter_chunk(1)
          return carry

        lax.fori_loop(0, _NCHUNK // 2, _pair, 0)
        _waitin(0)

  return grid_scatter


_SC_CALL = _build_sc_call()


def kernel(voxel_features, indices):
  idx = indices.astype(jnp.int32)
  npad = _NPAD - _N
  iz = jnp.concatenate([idx[:, 0], jnp.full((npad,), _G, jnp.int32)])
  iy = jnp.concatenate([idx[:, 1], jnp.zeros((npad,), jnp.int32)])
  ix = jnp.concatenate([idx[:, 2], jnp.zeros((npad,), jnp.int32)])
  feats = jnp.concatenate(
      [voxel_features.astype(jnp.float32),
       jnp.zeros((npad, _CH), jnp.float32)])
  out = _SC_CALL(iz, iy, ix, feats)
  return out[:_ROWS].reshape(_G, _G, _G, _CH)
